# Initial kernel scaffold; baseline (speedup 1.0000x reference)
import jax
import jax.numpy as jnp
from jax.experimental import pallas as pl


def kernel(x, edge_index0, edge_index1, W_self1, W_neigh1, b1, W_self2, W_neigh2, b2):
    raise NotImplementedError("write your pallas kernel here")


# speedup vs baseline: 6.7470x; 6.7470x over previous
import functools

import jax
import jax.numpy as jnp
from jax import lax
from jax.experimental import pallas as pl
from jax.experimental.pallas import tpu as pltpu
from jax.experimental.pallas import tpu_sc as plsc

N = 10000
D = 128
DP = 144
E = 320000

NC = 2
NS = 16
NW = NC * NS
EPW = E // NW
CHUNK = 125
NCHUNK = EPW // CHUNK
ROWS_PER_TILE = N // NS
RCHUNK = ROWS_PER_TILE // CHUNK

_MESH = plsc.VectorSubcoreMesh(core_axis_name="c", subcore_axis_name="s")


@functools.partial(
    pl.kernel,
    out_type=jax.ShapeDtypeStruct((NC, N, DP), jnp.float32),
    mesh=_MESH,
    compiler_params=pltpu.CompilerParams(use_tc_tiling_on_sc=False),
    scratch_types=[
        pltpu.VMEM((NCHUNK, CHUNK), jnp.int32),
        pltpu.VMEM((NCHUNK, CHUNK), jnp.int32),
        pltpu.VMEM((CHUNK, DP), jnp.float32),
        pltpu.VMEM_SHARED((N, DP), jnp.float32),
        pltpu.SemaphoreType.DMA,
    ],
)
def _sc_aggregate(table_hbm, src_hbm, dst_hbm, zeros_hbm, out_hbm,
                  src_v, dst_v, rows_v, acc_sh, sem):
    c = lax.axis_index("c")
    s = lax.axis_index("s")
    w = c * NS + s
    r0 = s * ROWS_PER_TILE
    pltpu.sync_copy(zeros_hbm.at[pl.ds(r0, ROWS_PER_TILE)],
                    acc_sh.at[pl.ds(r0, ROWS_PER_TILE)])
    pltpu.sync_copy(src_hbm.at[pl.ds(w * NCHUNK, NCHUNK)], src_v)
    pltpu.sync_copy(dst_hbm.at[pl.ds(w * NCHUNK, NCHUNK)], dst_v)
    plsc.subcore_barrier()

    def body(j, carry):
        pltpu.async_copy(table_hbm.at[src_v.at[j]], rows_v, sem).wait()
        pltpu.sync_copy(rows_v, acc_sh.at[dst_v.at[j]], add=True)
        return carry

    lax.fori_loop(0, NCHUNK, body, 0)
    plsc.subcore_barrier()
    pltpu.sync_copy(acc_sh.at[pl.ds(r0, ROWS_PER_TILE)],
                    out_hbm.at[c, pl.ds(r0, ROWS_PER_TILE)])


_RBLK = 1000


def _dense_body(relu, aug_out, h_ref, p0_ref, p1_ref, ws_ref, wn_ref, b_ref,
                o_ref):
    h = h_ref[:, :D]
    ssum = p0_ref[...] + p1_ref[...]
    feat = ssum[:, :D]
    deg = jnp.maximum(ssum[:, D:D + 1], 1.0)
    hn = feat / deg
    act = (jnp.dot(h, ws_ref[...], preferred_element_type=jnp.float32)
           + jnp.dot(hn, wn_ref[...], preferred_element_type=jnp.float32)
           + b_ref[...])
    if relu:
        act = jnp.maximum(act, 0.0)
    if aug_out:
        o_ref[:, :D] = act
        lane = lax.broadcasted_iota(jnp.int32, (_RBLK, DP - D), 1)
        o_ref[:, D:] = jnp.where(lane == 0, 1.0, 0.0)
    else:
        o_ref[...] = act


def _dense(h_aug, p0, p1, w_self, w_neigh, b, relu, aug_out):
    odp = DP if aug_out else D
    grid = (N // _RBLK,)
    return pl.pallas_call(
        functools.partial(_dense_body, relu, aug_out),
        grid=grid,
        in_specs=[
            pl.BlockSpec((_RBLK, DP), lambda i: (i, 0)),
            pl.BlockSpec((_RBLK, DP), lambda i: (i, 0)),
            pl.BlockSpec((_RBLK, DP), lambda i: (i, 0)),
            pl.BlockSpec((D, D), lambda i: (0, 0)),
            pl.BlockSpec((D, D), lambda i: (0, 0)),
            pl.BlockSpec((1, D), lambda i: (0, 0)),
        ],
        out_specs=pl.BlockSpec((_RBLK, odp), lambda i: (i, 0)),
        out_shape=jax.ShapeDtypeStruct((N, odp), jnp.float32),
    )(h_aug, p0, p1, w_self, w_neigh, b)


def kernel(x, edge_index0, edge_index1, W_self1, W_neigh1, b1,
           W_self2, W_neigh2, b2):
    src0 = edge_index0[0].astype(jnp.int32).reshape(E // CHUNK, CHUNK)
    dst0 = edge_index0[1].astype(jnp.int32).reshape(E // CHUNK, CHUNK)
    src1 = edge_index1[0].astype(jnp.int32).reshape(E // CHUNK, CHUNK)
    dst1 = edge_index1[1].astype(jnp.int32).reshape(E // CHUNK, CHUNK)

    aug = jnp.zeros((N, DP - D), jnp.float32).at[:, 0].set(1.0)
    x_aug = jnp.concatenate([x, aug], axis=1)
    zeros = jnp.zeros((N, DP), jnp.float32)
    b1r = b1.reshape(1, D)
    b2r = b2.reshape(1, D)

    p_l1 = _sc_aggregate(x_aug, src0, dst0, zeros)
    h_aug = _dense(x_aug, p_l1[0], p_l1[1], W_self1, W_neigh1, b1r,
                   relu=True, aug_out=True)
    p_l2 = _sc_aggregate(h_aug, src1, dst1, zeros)
    out = _dense(h_aug, p_l2[0], p_l2[1], W_self2, W_neigh2, b2r,
                 relu=False, aug_out=False)
    return out


# speedup vs baseline: 9.1725x; 1.3595x over previous
import functools

import jax
import jax.numpy as jnp
from jax import lax
from jax.experimental import pallas as pl
from jax.experimental.pallas import tpu as pltpu
from jax.experimental.pallas import tpu_sc as plsc

N = 10000
D = 128
DP = 144
E = 320000

NC = 2
NS = 16
NW = NC * NS
EPW = E // NW
CHUNK = 125
NCHUNK = EPW // CHUNK
SUPER = 8
SUPN = NCHUNK // SUPER
ROWS_PER_TILE = N // NS
RCHUNK = ROWS_PER_TILE // CHUNK

_MESH = plsc.VectorSubcoreMesh(core_axis_name="c", subcore_axis_name="s")


@functools.partial(
    pl.kernel,
    out_type=jax.ShapeDtypeStruct((NC, N, DP), jnp.float32),
    mesh=_MESH,
    compiler_params=pltpu.CompilerParams(use_tc_tiling_on_sc=False),
    scratch_types=[
        pltpu.VMEM((SUPER, CHUNK), jnp.int32),
        pltpu.VMEM((SUPER, CHUNK), jnp.int32),
        pltpu.VMEM((SUPER, CHUNK), jnp.int32),
        pltpu.VMEM((SUPER, CHUNK), jnp.int32),
        pltpu.VMEM((CHUNK, DP), jnp.float32),
        pltpu.VMEM((CHUNK, DP), jnp.float32),
        pltpu.VMEM_SHARED((N, DP), jnp.float32),
        pltpu.SemaphoreType.DMA,
        pltpu.SemaphoreType.DMA,
        pltpu.SemaphoreType.DMA,
        pltpu.SemaphoreType.DMA,
    ],
)
def _sc_aggregate(table_hbm, src_hbm, dst_hbm, zeros_hbm, out_hbm,
                  src_a, dst_a, src_b, dst_b, rows0_v, rows1_v, acc_sh,
                  sem0, sem1, sem_is, sem_id):
    c = lax.axis_index("c")
    s = lax.axis_index("s")
    w = c * NS + s
    r0 = s * ROWS_PER_TILE
    base = w * NCHUNK
    pltpu.sync_copy(zeros_hbm.at[pl.ds(r0, ROWS_PER_TILE)],
                    acc_sh.at[pl.ds(r0, ROWS_PER_TILE)])
    pltpu.sync_copy(src_hbm.at[pl.ds(base, SUPER)], src_a)
    pltpu.sync_copy(dst_hbm.at[pl.ds(base, SUPER)], dst_a)
    plsc.subcore_barrier()

    def process_super(u, src_c, dst_c, src_n, dst_n):
        @pl.when(u + 1 < SUPN)
        def _():
            off = base + (u + 1) * SUPER
            pltpu.async_copy(src_hbm.at[pl.ds(off, SUPER)], src_n, sem_is)
            pltpu.async_copy(dst_hbm.at[pl.ds(off, SUPER)], dst_n, sem_id)

        pltpu.async_copy(table_hbm.at[src_c.at[0]], rows0_v, sem0)

        def inner(i, carry):
            j0 = 2 * i
            pltpu.async_copy(table_hbm.at[src_c.at[j0 + 1]], rows1_v, sem1)
            pltpu.make_async_copy(table_hbm.at[src_c.at[j0]], rows0_v,
                                  sem0).wait()
            pltpu.sync_copy(rows0_v, acc_sh.at[dst_c.at[j0]], add=True)

            @pl.when(j0 + 2 < SUPER)
            def _():
                pltpu.async_copy(table_hbm.at[src_c.at[j0 + 2]], rows0_v,
                                 sem0)

            pltpu.make_async_copy(table_hbm.at[src_c.at[j0 + 1]], rows1_v,
                                  sem1).wait()
            pltpu.sync_copy(rows1_v, acc_sh.at[dst_c.at[j0 + 1]], add=True)
            return carry

        lax.fori_loop(0, SUPER // 2, inner, 0)

        @pl.when(u + 1 < SUPN)
        def _():
            pltpu.make_async_copy(src_hbm.at[pl.ds(base, SUPER)], src_n,
                                  sem_is).wait()
            pltpu.make_async_copy(dst_hbm.at[pl.ds(base, SUPER)], dst_n,
                                  sem_id).wait()

    def super_body(t, carry):
        u0 = 2 * t
        process_super(u0, src_a, dst_a, src_b, dst_b)
        process_super(u0 + 1, src_b, dst_b, src_a, dst_a)
        return carry

    lax.fori_loop(0, SUPN // 2, super_body, 0)
    plsc.subcore_barrier()
    pltpu.sync_copy(acc_sh.at[pl.ds(r0, ROWS_PER_TILE)],
                    out_hbm.at[c, pl.ds(r0, ROWS_PER_TILE)])


_RBLK = 1000


def _dense_body(relu, aug_out, h_ref, p_ref, ws_ref, wn_ref, b_ref,
                o_ref):
    h = h_ref[:, :D]
    ssum = p_ref[0] + p_ref[1]
    feat = ssum[:, :D]
    deg = jnp.maximum(ssum[:, D:D + 1], 1.0)
    hn = feat / deg
    act = (jnp.dot(h, ws_ref[...], preferred_element_type=jnp.float32)
           + jnp.dot(hn, wn_ref[...], preferred_element_type=jnp.float32)
           + b_ref[...])
    if relu:
        act = jnp.maximum(act, 0.0)
    if aug_out:
        o_ref[:, :D] = act
        lane = lax.broadcasted_iota(jnp.int32, (_RBLK, DP - D), 1)
        o_ref[:, D:] = jnp.where(lane == 0, 1.0, 0.0)
    else:
        o_ref[...] = act


def _dense(h_aug, p, w_self, w_neigh, b, relu, aug_out):
    odp = DP if aug_out else D
    grid = (N // _RBLK,)
    return pl.pallas_call(
        functools.partial(_dense_body, relu, aug_out),
        grid=grid,
        in_specs=[
            pl.BlockSpec((_RBLK, DP), lambda i: (i, 0)),
            pl.BlockSpec((NC, _RBLK, DP), lambda i: (0, i, 0)),
            pl.BlockSpec((D, D), lambda i: (0, 0)),
            pl.BlockSpec((D, D), lambda i: (0, 0)),
            pl.BlockSpec((1, D), lambda i: (0, 0)),
        ],
        out_specs=pl.BlockSpec((_RBLK, odp), lambda i: (i, 0)),
        out_shape=jax.ShapeDtypeStruct((N, odp), jnp.float32),
    )(h_aug, p, w_self, w_neigh, b)


def kernel(x, edge_index0, edge_index1, W_self1, W_neigh1, b1,
           W_self2, W_neigh2, b2):
    src0 = edge_index0[0].astype(jnp.int32).reshape(E // CHUNK, CHUNK)
    dst0 = edge_index0[1].astype(jnp.int32).reshape(E // CHUNK, CHUNK)
    src1 = edge_index1[0].astype(jnp.int32).reshape(E // CHUNK, CHUNK)
    dst1 = edge_index1[1].astype(jnp.int32).reshape(E // CHUNK, CHUNK)

    aug = jnp.zeros((N, DP - D), jnp.float32).at[:, 0].set(1.0)
    x_aug = jnp.concatenate([x, aug], axis=1)
    zeros = jnp.zeros((N, DP), jnp.float32)
    b1r = b1.reshape(1, D)
    b2r = b2.reshape(1, D)

    p_l1 = _sc_aggregate(x_aug, src0, dst0, zeros)
    h_aug = _dense(x_aug, p_l1, W_self1, W_neigh1, b1r,
                   relu=True, aug_out=True)
    p_l2 = _sc_aggregate(h_aug, src1, dst1, zeros)
    out = _dense(h_aug, p_l2, W_self2, W_neigh2, b2r,
                 relu=False, aug_out=False)
    return out
